# Initial kernel scaffold; baseline (speedup 1.0000x reference)
import jax
import jax.numpy as jnp
from jax import lax
from jax.experimental import pallas as pl
from jax.experimental.pallas import tpu as pltpu
from jax.experimental.pallas import tpu_sc as plsc

_N = 10000
_E = 320000
_D = 128
_DO = 32
_G = 64

_NC = 2
_NS = 16
_NW = _NC * _NS
_EPW = _E // _NW
_K = 80
_NCH = _EPW // _K
_RPT = _N // _NS
_ZR = 125
_DW = 16

_sc_mesh = plsc.VectorSubcoreMesh(
    core_axis_name="c", subcore_axis_name="s",
    num_cores=_NC, num_subcores=_NS)


def _deg_body(dst_hbm, out_hbm, dst_v, ones_v, zb_v, deg_sh):
    cid = lax.axis_index("c")
    sid = lax.axis_index("s")
    wid = cid * _NS + sid

    @pl.loop(0, _K)
    def _(i):
        ones_v[i, :] = jnp.ones((_DW,), jnp.float32)

    @pl.loop(0, _RPT)
    def _(i):
        zb_v[i, :] = jnp.zeros((_DW,), jnp.float32)

    pltpu.sync_copy(zb_v, deg_sh.at[pl.ds(sid * _RPT, _RPT)])
    plsc.subcore_barrier()

    pltpu.sync_copy(dst_hbm.at[wid], dst_v)

    @pl.loop(0, _NCH)
    def _(j):
        pltpu.sync_copy(ones_v, deg_sh.at[dst_v.at[j]], add=True)

    plsc.subcore_barrier()
    pltpu.sync_copy(deg_sh.at[pl.ds(sid * _RPT, _RPT)],
                    out_hbm.at[cid, pl.ds(sid * _RPT, _RPT)])


_deg_call = pl.kernel(
    _deg_body,
    out_type=jax.ShapeDtypeStruct((_NC, _N, _DW), jnp.float32),
    mesh=_sc_mesh,
    scratch_types=[
        pltpu.VMEM((_NCH, _K), jnp.int32),
        pltpu.VMEM((_K, _DW), jnp.float32),
        pltpu.VMEM((_RPT, _DW), jnp.float32),
        pltpu.VMEM_SHARED((_N, _DW), jnp.float32),
    ],
)


def _agg_body(y_hbm, src_hbm, dst_hbm, out_hbm, src_v, dst_v, rows_v, zb_v,
              agg_sh):
    cid = lax.axis_index("c")
    sid = lax.axis_index("s")
    wid = cid * _NS + sid

    @pl.loop(0, _ZR)
    def _(i):
        for c in range(_D // 16):
            zb_v[i, pl.ds(c * 16, 16)] = jnp.zeros((16,), jnp.float32)

    for t in range(_RPT // _ZR):
        pltpu.sync_copy(zb_v, agg_sh.at[pl.ds(sid * _RPT + t * _ZR, _ZR)])
    plsc.subcore_barrier()

    pltpu.sync_copy(src_hbm.at[wid], src_v)
    pltpu.sync_copy(dst_hbm.at[wid], dst_v)

    @pl.loop(0, _NCH)
    def _(j):
        pltpu.sync_copy(y_hbm.at[src_v.at[j]], rows_v)
        pltpu.sync_copy(rows_v, agg_sh.at[dst_v.at[j]], add=True)

    plsc.subcore_barrier()
    pltpu.sync_copy(agg_sh.at[pl.ds(sid * _RPT, _RPT)],
                    out_hbm.at[cid, pl.ds(sid * _RPT, _RPT)])


_agg_call = pl.kernel(
    _agg_body,
    out_type=jax.ShapeDtypeStruct((_NC, _N, _D), jnp.float32),
    mesh=_sc_mesh,
    scratch_types=[
        pltpu.VMEM((_NCH, _K), jnp.int32),
        pltpu.VMEM((_NCH, _K), jnp.int32),
        pltpu.VMEM((_K, _D), jnp.float32),
        pltpu.VMEM((_ZR, _D), jnp.float32),
        pltpu.VMEM_SHARED((_N, _D), jnp.float32),
    ],
)


def _dinv_of(degp_ref):
    deg = degp_ref[0, :, 0:1] + degp_ref[1, :, 0:1] + 1.0
    return lax.rsqrt(deg)


def _tc_a_body(x_ref, w_ref, degp_ref, y_ref):
    dinv = _dinv_of(degp_ref)
    y_ref[...] = dinv * jnp.dot(x_ref[...], w_ref[...],
                                preferred_element_type=jnp.float32)


_tc_a_call = pl.pallas_call(
    _tc_a_body,
    out_shape=jax.ShapeDtypeStruct((_N, _D), jnp.float32),
)


def _tc_b_body(aggp_ref, y_ref, degp_ref, w_ref, b_ref, y2_ref):
    dinv = _dinv_of(degp_ref)
    agg = aggp_ref[0] + aggp_ref[1] + y_ref[...]
    h = jnp.maximum(dinv * agg + b_ref[...], 0.0)
    y2_ref[...] = dinv * jnp.dot(h, w_ref[...],
                                 preferred_element_type=jnp.float32)


_tc_b_call = pl.pallas_call(
    _tc_b_body,
    out_shape=jax.ShapeDtypeStruct((_N, _D), jnp.float32),
)


def _tc_c_body(aggp_ref, y_ref, degp_ref, b_ref, batch_ref, wfc_ref, bfc_ref,
               out_ref):
    dinv = _dinv_of(degp_ref)
    agg = aggp_ref[0] + aggp_ref[1] + y_ref[...]
    h = jnp.maximum(dinv * agg + b_ref[...], 0.0)
    bt = batch_ref[...]
    onehot = (lax.broadcasted_iota(jnp.int32, (_G, _N), 0) == bt
              ).astype(jnp.float32)
    pool = jnp.dot(onehot, h, preferred_element_type=jnp.float32)
    cnt = jnp.sum(onehot, axis=1, keepdims=True)
    pooled = pool / jnp.maximum(cnt, 1.0)
    out_ref[...] = jnp.dot(pooled, wfc_ref[...],
                           preferred_element_type=jnp.float32) + bfc_ref[...]


_tc_c_call = pl.pallas_call(
    _tc_c_body,
    out_shape=jax.ShapeDtypeStruct((_G, _DO), jnp.float32),
)


def kernel(x, edge_index, batch, W1, b1, W2, b2, Wfc, bfc):
    src3 = edge_index[0].reshape(_NW, _NCH, _K)
    dst3 = edge_index[1].reshape(_NW, _NCH, _K)
    deg_p = _deg_call(dst3)
    y1 = _tc_a_call(x, W1, deg_p)
    agg1 = _agg_call(y1, src3, dst3)
    y2 = _tc_b_call(agg1, y1, deg_p, W2, b1.reshape(1, _D))
    agg2 = _agg_call(y2, src3, dst3)
    return _tc_c_call(agg2, y2, deg_p, b2.reshape(1, _D),
                      batch.reshape(1, _N), Wfc, bfc.reshape(1, _DO))


# speedup vs baseline: 2.5155x; 2.5155x over previous
import jax
import jax.numpy as jnp
from jax import lax
from jax.experimental import pallas as pl
from jax.experimental.pallas import tpu as pltpu
from jax.experimental.pallas import tpu_sc as plsc

_N = 10000
_E = 320000
_D = 128
_DO = 32
_G = 64

_NC = 2
_NS = 16
_NW = _NC * _NS

_DK = 80
_DCPG = 25
_DNG = (_E // _NW) // (_DK * _DCPG)

_K = 128
_CPG = 25
_ANG = (_E // _NC) // (_K * _CPG)
_CW = 8
_TW = 16
_NT = _D // _TW

_sc_mesh = plsc.VectorSubcoreMesh(
    core_axis_name="c", subcore_axis_name="s",
    num_cores=_NC, num_subcores=_NS)


def _deg_body(dst_hbm, out_hbm, dst_v, acc_v):
    cid = lax.axis_index("c")
    sid = lax.axis_index("s")
    wid = cid * _NS + sid
    zeros = jnp.zeros((16,), jnp.float32)
    ones = jnp.ones((16,), jnp.float32)

    @pl.loop(0, _N // 16)
    def _(i):
        acc_v[pl.ds(i * 16, 16)] = zeros

    for g in range(_DNG):
        pltpu.sync_copy(dst_hbm.at[wid, g], dst_v)

        @pl.loop(0, _DCPG)
        def _(j):
            for g8 in range(_DK // 16):
                d16 = dst_v[j, pl.ds(g8 * 16, 16)]
                plsc.addupdate_scatter(acc_v, [d16], ones)

    pltpu.sync_copy(acc_v, out_hbm.at[wid])


_deg_call = pl.kernel(
    _deg_body,
    out_type=jax.ShapeDtypeStruct((_NW, _N), jnp.float32),
    mesh=_sc_mesh,
    compiler_params=pltpu.CompilerParams(
        needs_layout_passes=False, use_tc_tiling_on_sc=False),
    scratch_types=[
        pltpu.VMEM((_DCPG, _DK), jnp.int32),
        pltpu.VMEM((_N,), jnp.float32),
    ],
)


def _agg_body(yt_hbm, src_hbm, dst_hbm, out_hbm, src_v, dst_v, rblk_v, acc_v):
    cid = lax.axis_index("c")
    sid = lax.axis_index("s")
    zeros = jnp.zeros((16,), jnp.float32)
    i16 = lax.iota(jnp.int32, 16)
    colbase = (sid % 2) * _CW
    table = yt_hbm.at[sid // 2]

    @pl.loop(0, (_N * _CW) // 16)
    def _(i):
        acc_v[pl.ds(i * 16, 16)] = zeros

    for g in range(_ANG):
        pltpu.sync_copy(src_hbm.at[cid, g], src_v)
        pltpu.sync_copy(dst_hbm.at[cid, g], dst_v)

        @pl.loop(0, _CPG)
        def _(j):
            pltpu.sync_copy(table.at[src_v.at[j]], rblk_v)
            for g8 in range(_K // 16):
                d16 = dst_v[j, pl.ds(g8 * 16, 16)]
                base = d16 * _CW
                e16 = g8 * 16 + i16
                for c in range(_CW):
                    cvec = jnp.zeros((16,), jnp.int32) + (colbase + c)
                    vals = plsc.load_gather(rblk_v, [e16, cvec])
                    plsc.addupdate_scatter(acc_v, [base + c], vals)

    pltpu.sync_copy(acc_v, out_hbm.at[cid, sid])


_agg_call = pl.kernel(
    _agg_body,
    out_type=jax.ShapeDtypeStruct((_NC, _NS, _N * _CW), jnp.float32),
    mesh=_sc_mesh,
    compiler_params=pltpu.CompilerParams(
        needs_layout_passes=False, use_tc_tiling_on_sc=False),
    scratch_types=[
        pltpu.VMEM((_CPG, _K), jnp.int32),
        pltpu.VMEM((_CPG, _K), jnp.int32),
        pltpu.VMEM((_K, _TW), jnp.float32),
        pltpu.VMEM((_N * _CW,), jnp.float32),
    ],
)


def _dinv_of(degp_ref):
    ones32 = jnp.ones((_NW, 1), jnp.float32)
    deg = lax.dot_general(degp_ref[...], ones32, (((0,), (0,)), ((), ())),
                          preferred_element_type=jnp.float32) + 1.0
    return lax.rsqrt(deg)


def _tc_a_body(x_ref, w_ref, degp_ref, y_ref):
    dinv = _dinv_of(degp_ref)
    y_ref[...] = dinv * jnp.dot(x_ref[...], w_ref[...],
                                preferred_element_type=jnp.float32)


_tc_a_call = pl.pallas_call(
    _tc_a_body,
    out_shape=jax.ShapeDtypeStruct((_N, _D), jnp.float32),
)


def _tc_b_body(aggp_ref, y_ref, degp_ref, w_ref, b_ref, y2_ref):
    dinv = _dinv_of(degp_ref)
    y = y_ref[...]
    agg = aggp_ref[0] + aggp_ref[1]
    h = jnp.maximum(dinv * (agg + y) + b_ref[...], 0.0)
    y2_ref[...] = dinv * jnp.dot(h, w_ref[...],
                                 preferred_element_type=jnp.float32)


_tc_b_call = pl.pallas_call(
    _tc_b_body,
    out_shape=jax.ShapeDtypeStruct((_N, _D), jnp.float32),
)


def _tc_c_body(aggp_ref, y_ref, degp_ref, b_ref, batch_ref, wfc_ref,
               bfc_ref, out_ref):
    dinv = _dinv_of(degp_ref)
    y = y_ref[...]
    agg = aggp_ref[0] + aggp_ref[1]
    h = jnp.maximum(dinv * (agg + y) + b_ref[...], 0.0)
    bt = batch_ref[...]
    onehot = (lax.broadcasted_iota(jnp.int32, (_G, _N), 0) == bt
              ).astype(jnp.float32)
    pool = jnp.dot(onehot, h, preferred_element_type=jnp.float32)
    cnt = jnp.sum(onehot, axis=1, keepdims=True)
    pooled = pool / jnp.maximum(cnt, 1.0)
    out_ref[...] = jnp.dot(pooled, wfc_ref[...],
                           preferred_element_type=jnp.float32) + bfc_ref[...]


_tc_c_call = pl.pallas_call(
    _tc_c_body,
    out_shape=jax.ShapeDtypeStruct((_G, _DO), jnp.float32),
)


def _col_tables(y):
    return y.reshape(_N, _NT, _TW).transpose(1, 0, 2)


def _merge_partials(aggp):
    return aggp.reshape(_NC, _NS, _N, _CW).transpose(0, 2, 1, 3).reshape(
        _NC, _N, _D)


def kernel(x, edge_index, batch, W1, b1, W2, b2, Wfc, bfc):
    src_a = edge_index[0].reshape(_NC, _ANG, _CPG, _K)
    dst_a = edge_index[1].reshape(_NC, _ANG, _CPG, _K)
    dst_d = edge_index[1].reshape(_NW, _DNG, _DCPG, _DK)
    deg_p = _deg_call(dst_d)
    y1 = _tc_a_call(x, W1, deg_p)
    agg1 = _merge_partials(_agg_call(_col_tables(y1), src_a, dst_a))
    y2 = _tc_b_call(agg1, y1, deg_p, W2, b1.reshape(1, _D))
    agg2 = _merge_partials(_agg_call(_col_tables(y2), src_a, dst_a))
    return _tc_c_call(agg2, y2, deg_p, b2.reshape(1, _D),
                      batch.reshape(1, _N), Wfc, bfc.reshape(1, _DO))


# speedup vs baseline: 4.7674x; 1.8952x over previous
import jax
import jax.numpy as jnp
from jax import lax
from jax.experimental import pallas as pl
from jax.experimental.pallas import tpu as pltpu
from jax.experimental.pallas import tpu_sc as plsc

_N = 10000
_E = 320000
_D = 128
_DO = 32
_G = 64

_NC = 2
_NS = 16
_NW = _NC * _NS

_DK = 80
_DCPG = 25
_DNG = (_E // _NW) // (_DK * _DCPG)

_K = 128
_CPG = 25
_ANG = (_E // _NC) // (_K * _CPG)
_CW = 8
_TW = 16
_NT = _D // _TW

_sc_mesh = plsc.VectorSubcoreMesh(
    core_axis_name="c", subcore_axis_name="s",
    num_cores=_NC, num_subcores=_NS)


def _deg_body(dst_hbm, out_hbm, dst_v, acc_v):
    cid = lax.axis_index("c")
    sid = lax.axis_index("s")
    wid = cid * _NS + sid
    zeros = jnp.zeros((16,), jnp.float32)
    ones = jnp.ones((16,), jnp.float32)

    @pl.loop(0, _N // 16)
    def _(i):
        acc_v[pl.ds(i * 16, 16)] = zeros

    for g in range(_DNG):
        pltpu.sync_copy(dst_hbm.at[wid, g], dst_v)

        @pl.loop(0, _DCPG)
        def _(j):
            for g8 in range(_DK // 16):
                d16 = dst_v[j, pl.ds(g8 * 16, 16)]
                plsc.addupdate_scatter(acc_v, [d16], ones)

    pltpu.sync_copy(acc_v, out_hbm.at[wid])


_deg_call = pl.kernel(
    _deg_body,
    out_type=jax.ShapeDtypeStruct((_NW, _N), jnp.float32),
    mesh=_sc_mesh,
    compiler_params=pltpu.CompilerParams(
        needs_layout_passes=False, use_tc_tiling_on_sc=False),
    scratch_types=[
        pltpu.VMEM((_DCPG, _DK), jnp.int32),
        pltpu.VMEM((_N,), jnp.float32),
    ],
)


_NBUF = 5


def _agg_body(yt_hbm, src_hbm, dst_hbm, out_hbm, src_v, dst_v, rblk_v, acc_v,
              isem0, isem1, rsem0, rsem1, rsem2, rsem3, rsem4):
    cid = lax.axis_index("c")
    sid = lax.axis_index("s")
    zeros = jnp.zeros((16,), jnp.float32)
    i16 = lax.iota(jnp.int32, 16)
    colbase = (sid % 2) * _CW
    table = yt_hbm.at[sid // 2]
    isems = (isem0, isem1)
    rsems = (rsem0, rsem1, rsem2, rsem3, rsem4)
    cvecs = [jnp.zeros((16,), jnp.int32) + (colbase + c) for c in range(_CW)]
    e16s = [g8 * 16 + i16 for g8 in range(_K // 16)]

    @pl.loop(0, (_N * _CW) // 16)
    def _(i):
        acc_v[pl.ds(i * 16, 16)] = zeros

    def fire_idx(g, ib):
        pltpu.async_copy(src_hbm.at[cid, g], src_v.at[ib], isems[ib])
        pltpu.async_copy(dst_hbm.at[cid, g], dst_v.at[ib], isems[ib])

    def wait_idx(g, ib):
        pltpu.make_async_copy(src_hbm.at[cid, g], src_v.at[ib],
                              isems[ib]).wait()
        pltpu.make_async_copy(dst_hbm.at[cid, g], dst_v.at[ib],
                              isems[ib]).wait()

    def fire_rows(ib, j, b):
        pltpu.async_copy(table.at[src_v.at[ib, j]], rblk_v.at[b], rsems[b])

    def wait_rows(ib, j, b):
        pltpu.make_async_copy(table.at[src_v.at[ib, j]], rblk_v.at[b],
                              rsems[b]).wait()

    def process(ib, j, b):
        for g8 in range(_K // 16):
            d16 = dst_v[ib, j, pl.ds(g8 * 16, 16)]
            base = d16 * _CW
            for c in range(_CW):
                vals = plsc.load_gather(rblk_v.at[b], [e16s[g8], cvecs[c]])
                plsc.addupdate_scatter(acc_v, [base + c], vals)

    pltpu.sync_copy(src_hbm.at[cid, 0], src_v.at[0])
    pltpu.sync_copy(dst_hbm.at[cid, 0], dst_v.at[0])

    @pl.loop(0, _ANG, step=2)
    def _(gg):
        for ib in range(2):
            g = gg + ib

            @pl.when(g > 0)
            def _():
                wait_idx(g, ib)

            @pl.when(g + 1 < _ANG)
            def _():
                fire_idx(g + 1, 1 - ib)

            for b in range(_NBUF):
                fire_rows(ib, b, b)

            @pl.loop(0, _CPG, step=_NBUF)
            def _(jb):
                for b in range(_NBUF):
                    j = jb + b
                    wait_rows(ib, j, b)
                    process(ib, j, b)

                    @pl.when(j + _NBUF < _CPG)
                    def _():
                        fire_rows(ib, j + _NBUF, b)

    pltpu.sync_copy(acc_v, out_hbm.at[cid, sid])


_agg_call = pl.kernel(
    _agg_body,
    out_type=jax.ShapeDtypeStruct((_NC, _NS, _N * _CW), jnp.float32),
    mesh=_sc_mesh,
    compiler_params=pltpu.CompilerParams(
        needs_layout_passes=False, use_tc_tiling_on_sc=False),
    scratch_types=[
        pltpu.VMEM((2, _CPG, _K), jnp.int32),
        pltpu.VMEM((2, _CPG, _K), jnp.int32),
        pltpu.VMEM((_NBUF, _K, _TW), jnp.float32),
        pltpu.VMEM((_N * _CW,), jnp.float32),
        pltpu.SemaphoreType.DMA,
        pltpu.SemaphoreType.DMA,
        pltpu.SemaphoreType.DMA,
        pltpu.SemaphoreType.DMA,
        pltpu.SemaphoreType.DMA,
        pltpu.SemaphoreType.DMA,
        pltpu.SemaphoreType.DMA,
    ],
)


def _dinv_of(degp_ref):
    ones32 = jnp.ones((_NW, 1), jnp.float32)
    deg = lax.dot_general(degp_ref[...], ones32, (((0,), (0,)), ((), ())),
                          preferred_element_type=jnp.float32) + 1.0
    return lax.rsqrt(deg)


def _tc_a_body(x_ref, w_ref, degp_ref, y_ref):
    dinv = _dinv_of(degp_ref)
    y_ref[...] = dinv * jnp.dot(x_ref[...], w_ref[...],
                                preferred_element_type=jnp.float32)


_tc_a_call = pl.pallas_call(
    _tc_a_body,
    out_shape=jax.ShapeDtypeStruct((_N, _D), jnp.float32),
)


def _tc_b_body(aggp_ref, y_ref, degp_ref, w_ref, b_ref, y2_ref):
    dinv = _dinv_of(degp_ref)
    y = y_ref[...]
    agg = aggp_ref[0] + aggp_ref[1]
    h = jnp.maximum(dinv * (agg + y) + b_ref[...], 0.0)
    y2_ref[...] = dinv * jnp.dot(h, w_ref[...],
                                 preferred_element_type=jnp.float32)


_tc_b_call = pl.pallas_call(
    _tc_b_body,
    out_shape=jax.ShapeDtypeStruct((_N, _D), jnp.float32),
)


def _tc_c_body(aggp_ref, y_ref, degp_ref, b_ref, batch_ref, wfc_ref,
               bfc_ref, out_ref):
    dinv = _dinv_of(degp_ref)
    y = y_ref[...]
    agg = aggp_ref[0] + aggp_ref[1]
    h = jnp.maximum(dinv * (agg + y) + b_ref[...], 0.0)
    bt = batch_ref[...]
    onehot = (lax.broadcasted_iota(jnp.int32, (_G, _N), 0) == bt
              ).astype(jnp.float32)
    pool = jnp.dot(onehot, h, preferred_element_type=jnp.float32)
    cnt = jnp.sum(onehot, axis=1, keepdims=True)
    pooled = pool / jnp.maximum(cnt, 1.0)
    out_ref[...] = jnp.dot(pooled, wfc_ref[...],
                           preferred_element_type=jnp.float32) + bfc_ref[...]


_tc_c_call = pl.pallas_call(
    _tc_c_body,
    out_shape=jax.ShapeDtypeStruct((_G, _DO), jnp.float32),
)


def _col_tables(y):
    return y.reshape(_N, _NT, _TW).transpose(1, 0, 2)


def _merge_partials(aggp):
    return aggp.reshape(_NC, _NS, _N, _CW).transpose(0, 2, 1, 3).reshape(
        _NC, _N, _D)


def kernel(x, edge_index, batch, W1, b1, W2, b2, Wfc, bfc):
    src_a = edge_index[0].reshape(_NC, _ANG, _CPG, _K)
    dst_a = edge_index[1].reshape(_NC, _ANG, _CPG, _K)
    dst_d = edge_index[1].reshape(_NW, _DNG, _DCPG, _DK)
    deg_p = _deg_call(dst_d)
    y1 = _tc_a_call(x, W1, deg_p)
    agg1 = _merge_partials(_agg_call(_col_tables(y1), src_a, dst_a))
    y2 = _tc_b_call(agg1, y1, deg_p, W2, b1.reshape(1, _D))
    agg2 = _merge_partials(_agg_call(_col_tables(y2), src_a, dst_a))
    return _tc_c_call(agg2, y2, deg_p, b2.reshape(1, _D),
                      batch.reshape(1, _N), Wfc, bfc.reshape(1, _DO))


# speedup vs baseline: 8.0525x; 1.6891x over previous
import jax
import jax.numpy as jnp
from jax import lax
from jax.experimental import pallas as pl
from jax.experimental.pallas import tpu as pltpu
from jax.experimental.pallas import tpu_sc as plsc

_N = 10000
_E = 320000
_D = 128
_DO = 32
_G = 64

_NC = 2
_NS = 16
_NW = _NC * _NS

_DK = 80
_DCPG = 25
_DNG = (_E // _NW) // (_DK * _DCPG)

_K = 128
_CPG = 25
_ANG = (_E // _NC) // (_K * _CPG)
_CW = 8
_TW = 16
_NT = _D // _TW

_sc_mesh = plsc.VectorSubcoreMesh(
    core_axis_name="c", subcore_axis_name="s",
    num_cores=_NC, num_subcores=_NS)


def _deg_body(dst_hbm, out_hbm, dst_v, acc_v):
    cid = lax.axis_index("c")
    sid = lax.axis_index("s")
    wid = cid * _NS + sid
    zeros = jnp.zeros((16,), jnp.float32)
    ones = jnp.ones((16,), jnp.float32)

    @pl.loop(0, _N // 16)
    def _(i):
        acc_v[pl.ds(i * 16, 16)] = zeros

    for g in range(_DNG):
        pltpu.sync_copy(dst_hbm.at[wid, g], dst_v)

        @pl.loop(0, _DCPG)
        def _(j):
            for g8 in range(_DK // 16):
                d16 = dst_v[j, pl.ds(g8 * 16, 16)]
                plsc.addupdate_scatter(acc_v, [d16], ones)

    pltpu.sync_copy(acc_v, out_hbm.at[wid])


_deg_call = pl.kernel(
    _deg_body,
    out_type=jax.ShapeDtypeStruct((_NW, _N), jnp.float32),
    mesh=_sc_mesh,
    compiler_params=pltpu.CompilerParams(
        needs_layout_passes=False, use_tc_tiling_on_sc=False),
    scratch_types=[
        pltpu.VMEM((_DCPG, _DK), jnp.int32),
        pltpu.VMEM((_N,), jnp.float32),
    ],
)


_NBUF = 5


def _agg_body(yt_hbm, src_hbm, dst_hbm, out_hbm, src_v, dst_v, rblk_v, acc_v,
              isem0, isem1, rsem0, rsem1, rsem2, rsem3, rsem4):
    cid = lax.axis_index("c")
    sid = lax.axis_index("s")
    zeros = jnp.zeros((16,), jnp.float32)
    i16 = lax.iota(jnp.int32, 16)
    colbase = (sid % 2) * _CW
    table = yt_hbm.at[sid // 2]
    isems = (isem0, isem1)
    rsems = (rsem0, rsem1, rsem2, rsem3, rsem4)
    crot = [(i16 + s) % _CW for s in range(_CW)]
    ccol = [colbase + c for c in crot]
    e16s = [g8 * 16 + i16 for g8 in range(_K // 16)]
    crotN = [c * _N for c in crot]

    @pl.loop(0, (_N * _CW) // 16)
    def _(i):
        acc_v[pl.ds(i * 16, 16)] = zeros

    def fire_idx(g, ib):
        pltpu.async_copy(src_hbm.at[cid, g], src_v.at[ib], isems[ib])
        pltpu.async_copy(dst_hbm.at[cid, g], dst_v.at[ib], isems[ib])

    def wait_idx(g, ib):
        pltpu.make_async_copy(src_hbm.at[cid, g], src_v.at[ib],
                              isems[ib]).wait()
        pltpu.make_async_copy(dst_hbm.at[cid, g], dst_v.at[ib],
                              isems[ib]).wait()

    def fire_rows(ib, j, b):
        pltpu.async_copy(table.at[src_v.at[ib, j]], rblk_v.at[b], rsems[b])

    def wait_rows(ib, j, b):
        pltpu.make_async_copy(table.at[src_v.at[ib, j]], rblk_v.at[b],
                              rsems[b]).wait()

    def process(ib, j, b):
        rflat = rblk_v.at[b]
        for g8 in range(_K // 16):
            d16 = dst_v[ib, j, pl.ds(g8 * 16, 16)]
            for s in range(_CW):
                vals = plsc.load_gather(rflat, [e16s[g8], ccol[s]])
                plsc.addupdate_scatter(acc_v, [crotN[s] + d16], vals)

    pltpu.sync_copy(src_hbm.at[cid, 0], src_v.at[0])
    pltpu.sync_copy(dst_hbm.at[cid, 0], dst_v.at[0])

    @pl.loop(0, _ANG, step=2)
    def _(gg):
        for ib in range(2):
            g = gg + ib

            @pl.when(g > 0)
            def _():
                wait_idx(g, ib)

            @pl.when(g + 1 < _ANG)
            def _():
                fire_idx(g + 1, 1 - ib)

            for b in range(_NBUF):
                fire_rows(ib, b, b)

            @pl.loop(0, _CPG, step=_NBUF)
            def _(jb):
                for b in range(_NBUF):
                    j = jb + b
                    wait_rows(ib, j, b)
                    process(ib, j, b)

                    @pl.when(j + _NBUF < _CPG)
                    def _():
                        fire_rows(ib, j + _NBUF, b)

    pltpu.sync_copy(acc_v, out_hbm.at[cid, sid])


_agg_call = pl.kernel(
    _agg_body,
    out_type=jax.ShapeDtypeStruct((_NC, _NS, _N * _CW), jnp.float32),
    mesh=_sc_mesh,
    compiler_params=pltpu.CompilerParams(
        needs_layout_passes=False, use_tc_tiling_on_sc=False),
    scratch_types=[
        pltpu.VMEM((2, _CPG, _K), jnp.int32),
        pltpu.VMEM((2, _CPG, _K), jnp.int32),
        pltpu.VMEM((_NBUF, _K, _TW), jnp.float32),
        pltpu.VMEM((_N * _CW,), jnp.float32),
        pltpu.SemaphoreType.DMA,
        pltpu.SemaphoreType.DMA,
        pltpu.SemaphoreType.DMA,
        pltpu.SemaphoreType.DMA,
        pltpu.SemaphoreType.DMA,
        pltpu.SemaphoreType.DMA,
        pltpu.SemaphoreType.DMA,
    ],
)


def _dinv_of(degp_ref):
    ones32 = jnp.ones((_NW, 1), jnp.float32)
    deg = lax.dot_general(degp_ref[...], ones32, (((0,), (0,)), ((), ())),
                          preferred_element_type=jnp.float32) + 1.0
    return lax.rsqrt(deg)


def _tc_a_body(x_ref, w_ref, degp_ref, y_ref):
    dinv = _dinv_of(degp_ref)
    y_ref[...] = dinv * jnp.dot(x_ref[...], w_ref[...],
                                preferred_element_type=jnp.float32)


_tc_a_call = pl.pallas_call(
    _tc_a_body,
    out_shape=jax.ShapeDtypeStruct((_N, _D), jnp.float32),
)


def _tc_b_body(aggp_ref, y_ref, degp_ref, w_ref, b_ref, y2_ref):
    dinv = _dinv_of(degp_ref)
    y = y_ref[...]
    agg = aggp_ref[0] + aggp_ref[1]
    h = jnp.maximum(dinv * (agg + y) + b_ref[...], 0.0)
    y2_ref[...] = dinv * jnp.dot(h, w_ref[...],
                                 preferred_element_type=jnp.float32)


_tc_b_call = pl.pallas_call(
    _tc_b_body,
    out_shape=jax.ShapeDtypeStruct((_N, _D), jnp.float32),
)


def _tc_c_body(aggp_ref, y_ref, degp_ref, b_ref, batch_ref, wfc_ref,
               bfc_ref, out_ref):
    dinv = _dinv_of(degp_ref)
    y = y_ref[...]
    agg = aggp_ref[0] + aggp_ref[1]
    h = jnp.maximum(dinv * (agg + y) + b_ref[...], 0.0)
    bt = batch_ref[...]
    onehot = (lax.broadcasted_iota(jnp.int32, (_G, _N), 0) == bt
              ).astype(jnp.float32)
    pool = jnp.dot(onehot, h, preferred_element_type=jnp.float32)
    cnt = jnp.sum(onehot, axis=1, keepdims=True)
    pooled = pool / jnp.maximum(cnt, 1.0)
    out_ref[...] = jnp.dot(pooled, wfc_ref[...],
                           preferred_element_type=jnp.float32) + bfc_ref[...]


_tc_c_call = pl.pallas_call(
    _tc_c_body,
    out_shape=jax.ShapeDtypeStruct((_G, _DO), jnp.float32),
)


def _col_tables(y):
    return y.reshape(_N, _NT, _TW).transpose(1, 0, 2)


def _merge_partials(aggp):
    return aggp.reshape(_NC, _NS, _CW, _N).transpose(0, 3, 1, 2).reshape(
        _NC, _N, _D)


def kernel(x, edge_index, batch, W1, b1, W2, b2, Wfc, bfc):
    src_a = edge_index[0].reshape(_NC, _ANG, _CPG, _K)
    dst_a = edge_index[1].reshape(_NC, _ANG, _CPG, _K)
    dst_d = edge_index[1].reshape(_NW, _DNG, _DCPG, _DK)
    deg_p = _deg_call(dst_d)
    y1 = _tc_a_call(x, W1, deg_p)
    agg1 = _merge_partials(_agg_call(_col_tables(y1), src_a, dst_a))
    y2 = _tc_b_call(agg1, y1, deg_p, W2, b1.reshape(1, _D))
    agg2 = _merge_partials(_agg_call(_col_tables(y2), src_a, dst_a))
    return _tc_c_call(agg2, y2, deg_p, b2.reshape(1, _D),
                      batch.reshape(1, _N), Wfc, bfc.reshape(1, _DO))


# speedup vs baseline: 8.5308x; 1.0594x over previous
import jax
import jax.numpy as jnp
from jax import lax
from jax.experimental import pallas as pl
from jax.experimental.pallas import tpu as pltpu
from jax.experimental.pallas import tpu_sc as plsc

_N = 10000
_E = 320000
_D = 128
_DO = 32
_G = 64

_NC = 2
_NS = 16
_NW = _NC * _NS

_DK = 80
_DCPG = 25
_DNG = (_E // _NW) // (_DK * _DCPG)

_K = 128
_CPG = 25
_ANG = (_E // _NC) // (_K * _CPG)
_CW = 8
_TW = 16
_NT = _D // _TW

_sc_mesh = plsc.VectorSubcoreMesh(
    core_axis_name="c", subcore_axis_name="s",
    num_cores=_NC, num_subcores=_NS)


def _deg_body(dst_hbm, out_hbm, dst_v, acc_v):
    cid = lax.axis_index("c")
    sid = lax.axis_index("s")
    wid = cid * _NS + sid
    zeros = jnp.zeros((16,), jnp.float32)
    ones = jnp.ones((16,), jnp.float32)

    @pl.loop(0, _N // 16)
    def _(i):
        acc_v[pl.ds(i * 16, 16)] = zeros

    for g in range(_DNG):
        pltpu.sync_copy(dst_hbm.at[wid, g], dst_v)

        @pl.loop(0, _DCPG)
        def _(j):
            for g8 in range(_DK // 16):
                d16 = dst_v[j, pl.ds(g8 * 16, 16)]
                plsc.addupdate_scatter(acc_v, [d16], ones)

    pltpu.sync_copy(acc_v, out_hbm.at[wid])


_deg_call = pl.kernel(
    _deg_body,
    out_type=jax.ShapeDtypeStruct((_NW, _N), jnp.float32),
    mesh=_sc_mesh,
    compiler_params=pltpu.CompilerParams(
        needs_layout_passes=False, use_tc_tiling_on_sc=False),
    scratch_types=[
        pltpu.VMEM((_DCPG, _DK), jnp.int32),
        pltpu.VMEM((_N,), jnp.float32),
    ],
)


_NBUF = 5


def _agg_body(yt_hbm, src_hbm, dst_hbm, out_hbm, src_v, dst_v, rblk_v, acc_v,
              isem, rsem0, rsem1, rsem2, rsem3, rsem4):
    cid = lax.axis_index("c")
    sid = lax.axis_index("s")
    zeros = jnp.zeros((16,), jnp.float32)
    i16 = lax.iota(jnp.int32, 16)
    colbase = (sid % 2) * _CW
    table = yt_hbm.at[sid // 2]
    rsems = (rsem0, rsem1, rsem2, rsem3, rsem4)
    crot = [(i16 + s) % _CW for s in range(_CW)]
    ccol = [colbase + c for c in crot]
    e16s = [g8 * 16 + i16 for g8 in range(_K // 16)]
    crotN = [c * _N for c in crot]
    nchunks = _ANG * _CPG

    @pl.loop(0, (_N * _CW) // 16)
    def _(i):
        acc_v[pl.ds(i * 16, 16)] = zeros

    def fire_idx(g):
        pltpu.async_copy(src_hbm.at[cid, g], src_v.at[g % 3], isem)
        pltpu.async_copy(dst_hbm.at[cid, g], dst_v.at[g % 3], isem)

    def wait_idx(g):
        pltpu.make_async_copy(src_hbm.at[cid, g], src_v.at[g % 3],
                              isem).wait()
        pltpu.make_async_copy(dst_hbm.at[cid, g], dst_v.at[g % 3],
                              isem).wait()

    def fire_rows(j, b):
        pltpu.async_copy(table.at[src_v.at[(j // _CPG) % 3, j % _CPG]],
                         rblk_v.at[b], rsems[b])

    def wait_rows(j, b):
        pltpu.make_async_copy(table.at[src_v.at[(j // _CPG) % 3, j % _CPG]],
                              rblk_v.at[b], rsems[b]).wait()

    def process(j, b):
        slot = (j // _CPG) % 3
        jin = j % _CPG
        rflat = rblk_v.at[b]
        for g8 in range(_K // 16):
            d16 = dst_v[slot, jin, pl.ds(g8 * 16, 16)]
            for s in range(_CW):
                vals = plsc.load_gather(rflat, [e16s[g8], ccol[s]])
                plsc.addupdate_scatter(acc_v, [crotN[s] + d16], vals)

    pltpu.sync_copy(src_hbm.at[cid, 0], src_v.at[0])
    pltpu.sync_copy(dst_hbm.at[cid, 0], dst_v.at[0])
    fire_idx(1)
    for b in range(_NBUF):
        fire_rows(b, b)

    @pl.loop(0, nchunks, step=_NBUF)
    def _(j0):
        g0 = j0 // _CPG

        @pl.when(jnp.logical_and(j0 % _CPG == 0, g0 + 2 < _ANG))
        def _():
            fire_idx(g0 + 2)

        for b in range(_NBUF):
            j = j0 + b
            wait_rows(j, b)
            process(j, b)
            jn = j + _NBUF

            if b == 0:
                @pl.when(jnp.logical_and(j0 % _CPG == _CPG - _NBUF,
                                         g0 + 1 < _ANG))
                def _():
                    wait_idx(g0 + 1)

            @pl.when(jn < nchunks)
            def _():
                fire_rows(jn, b)

    pltpu.sync_copy(acc_v, out_hbm.at[cid, sid])


_agg_call = pl.kernel(
    _agg_body,
    out_type=jax.ShapeDtypeStruct((_NC, _NS, _N * _CW), jnp.float32),
    mesh=_sc_mesh,
    compiler_params=pltpu.CompilerParams(
        needs_layout_passes=False, use_tc_tiling_on_sc=False),
    scratch_types=[
        pltpu.VMEM((3, _CPG, _K), jnp.int32),
        pltpu.VMEM((3, _CPG, _K), jnp.int32),
        pltpu.VMEM((_NBUF, _K, _TW), jnp.float32),
        pltpu.VMEM((_N * _CW,), jnp.float32),
        pltpu.SemaphoreType.DMA,
        pltpu.SemaphoreType.DMA,
        pltpu.SemaphoreType.DMA,
        pltpu.SemaphoreType.DMA,
        pltpu.SemaphoreType.DMA,
        pltpu.SemaphoreType.DMA,
    ],
)


def _dinv_of(degp_ref):
    ones32 = jnp.ones((_NW, 1), jnp.float32)
    deg = lax.dot_general(degp_ref[...], ones32, (((0,), (0,)), ((), ())),
                          preferred_element_type=jnp.float32) + 1.0
    return lax.rsqrt(deg)


def _tc_a_body(x_ref, w_ref, degp_ref, y_ref):
    dinv = _dinv_of(degp_ref)
    y_ref[...] = dinv * jnp.dot(x_ref[...], w_ref[...],
                                preferred_element_type=jnp.float32)


_tc_a_call = pl.pallas_call(
    _tc_a_body,
    out_shape=jax.ShapeDtypeStruct((_N, _D), jnp.float32),
)


def _tc_b_body(aggp_ref, y_ref, degp_ref, w_ref, b_ref, y2_ref):
    dinv = _dinv_of(degp_ref)
    y = y_ref[...]
    agg = aggp_ref[0] + aggp_ref[1]
    h = jnp.maximum(dinv * (agg + y) + b_ref[...], 0.0)
    y2_ref[...] = dinv * jnp.dot(h, w_ref[...],
                                 preferred_element_type=jnp.float32)


_tc_b_call = pl.pallas_call(
    _tc_b_body,
    out_shape=jax.ShapeDtypeStruct((_N, _D), jnp.float32),
)


def _tc_c_body(aggp_ref, y_ref, degp_ref, b_ref, batch_ref, wfc_ref,
               bfc_ref, out_ref):
    dinv = _dinv_of(degp_ref)
    y = y_ref[...]
    agg = aggp_ref[0] + aggp_ref[1]
    h = jnp.maximum(dinv * (agg + y) + b_ref[...], 0.0)
    bt = batch_ref[...]
    onehot = (lax.broadcasted_iota(jnp.int32, (_G, _N), 0) == bt
              ).astype(jnp.float32)
    pool = jnp.dot(onehot, h, preferred_element_type=jnp.float32)
    cnt = jnp.sum(onehot, axis=1, keepdims=True)
    pooled = pool / jnp.maximum(cnt, 1.0)
    out_ref[...] = jnp.dot(pooled, wfc_ref[...],
                           preferred_element_type=jnp.float32) + bfc_ref[...]


_tc_c_call = pl.pallas_call(
    _tc_c_body,
    out_shape=jax.ShapeDtypeStruct((_G, _DO), jnp.float32),
)


def _col_tables(y):
    return y.reshape(_N, _NT, _TW).transpose(1, 0, 2)


def _merge_partials(aggp):
    return aggp.reshape(_NC, _NS, _CW, _N).transpose(0, 3, 1, 2).reshape(
        _NC, _N, _D)


def kernel(x, edge_index, batch, W1, b1, W2, b2, Wfc, bfc):
    src_a = edge_index[0].reshape(_NC, _ANG, _CPG, _K)
    dst_a = edge_index[1].reshape(_NC, _ANG, _CPG, _K)
    dst_d = edge_index[1].reshape(_NW, _DNG, _DCPG, _DK)
    deg_p = _deg_call(dst_d)
    y1 = _tc_a_call(x, W1, deg_p)
    agg1 = _merge_partials(_agg_call(_col_tables(y1), src_a, dst_a))
    y2 = _tc_b_call(agg1, y1, deg_p, W2, b1.reshape(1, _D))
    agg2 = _merge_partials(_agg_call(_col_tables(y2), src_a, dst_a))
    return _tc_c_call(agg2, y2, deg_p, b2.reshape(1, _D),
                      batch.reshape(1, _N), Wfc, bfc.reshape(1, _DO))
